# Initial kernel scaffold; baseline (speedup 1.0000x reference)
import jax
import jax.numpy as jnp
from jax.experimental import pallas as pl


def kernel(x, edge_index, w1_miu, w1_sigma, w2_miu, w2_sigma):
    raise NotImplementedError("write your pallas kernel here")


# speedup vs baseline: 6.1124x; 6.1124x over previous
import functools

import jax
import jax.numpy as jnp
from jax import lax
from jax.experimental import pallas as pl
from jax.experimental.pallas import tpu as pltpu
from jax.experimental.pallas import tpu_sc as plsc

F32 = jnp.float32
NC = 2
NS = 16
LW = 128
DW = 16


def _sc_mesh():
    return plsc.VectorSubcoreMesh(
        core_axis_name="c", subcore_axis_name="s", num_cores=NC, num_subcores=NS
    )


def _build_deg(n_nodes, n_rows, k):
    rows_per_worker = n_rows // (NC * NS)
    steps = rows_per_worker // k
    ort = n_nodes // NS
    acc_rows = n_nodes

    def body(src_hbm, ones_hbm, zeros_hbm, out_hbm, idx_v, ones_v, tmp_v,
             acc_sh, sem):
        c = lax.axis_index("c")
        s = lax.axis_index("s")
        wid = s * NC + c
        pltpu.sync_copy(zeros_hbm.at[pl.ds(s * ort, ort)], tmp_v)
        pltpu.sync_copy(tmp_v, acc_sh.at[pl.ds(s * ort, ort)])
        pltpu.sync_copy(ones_hbm, ones_v)
        plsc.subcore_barrier()

        def step(i, carry):
            row0 = wid * rows_per_worker + i * k
            pltpu.sync_copy(src_hbm.at[pl.ds(row0, k)], idx_v)
            for j in range(k):
                pltpu.sync_copy(ones_v.at[pl.ds(j * LW, LW)],
                                acc_sh.at[idx_v.at[j]], add=True)
            return carry

        lax.fori_loop(0, steps, step, 0)
        plsc.subcore_barrier()
        pltpu.sync_copy(acc_sh.at[pl.ds(s * ort, ort)], tmp_v)
        pltpu.sync_copy(tmp_v, out_hbm.at[c, pl.ds(s * ort, ort)])
        plsc.subcore_barrier()

    return pl.kernel(
        body,
        out_type=jax.ShapeDtypeStruct((NC, n_nodes, DW), F32),
        mesh=_sc_mesh(),
        compiler_params=pltpu.CompilerParams(use_tc_tiling_on_sc=False),
        scratch_types=[
            pltpu.VMEM((k, LW), jnp.int32),
            pltpu.VMEM((k * LW, DW), F32),
            pltpu.VMEM((n_nodes // NS, DW), F32),
            pltpu.VMEM_SHARED((acc_rows, DW), F32),
            pltpu.SemaphoreType.DMA,
        ],
    )


def _build_spmm(n_nodes, n_rows, fc, nchunk, k, kg):
    rows_per_tile = n_rows // NS
    steps = rows_per_tile // k
    ngrp = k // kg
    ort = n_nodes // NS
    acc_rows = n_nodes

    def body(*refs):
        gs = refs[: 2 * nchunk]
        src_hbm, dst_hbm, zeros_hbm = refs[2 * nchunk: 2 * nchunk + 3]
        os_ = refs[2 * nchunk + 3: 4 * nchunk + 3]
        idxs_v, idxd_v, rows_v, acc_sh, sem = refs[4 * nchunk + 3:]
        c = lax.axis_index("c")
        s = lax.axis_index("s")

        def process(g, o):
            pltpu.sync_copy(zeros_hbm.at[pl.ds(s * ort, ort)],
                            rows_v.at[pl.ds(0, ort)])
            pltpu.sync_copy(rows_v.at[pl.ds(0, ort)],
                            acc_sh.at[pl.ds(s * ort, ort)])
            plsc.subcore_barrier()

            def step(i, carry):
                row0 = s * rows_per_tile + i * k
                pltpu.sync_copy(src_hbm.at[pl.ds(row0, k)], idxs_v)
                pltpu.sync_copy(dst_hbm.at[pl.ds(row0, k)], idxd_v)
                for gi in range(ngrp):
                    descs = [
                        pltpu.async_copy(g.at[idxd_v.at[gi * kg + j]],
                                         rows_v.at[pl.ds(j * LW, LW)], sem)
                        for j in range(kg)
                    ]
                    for d in descs:
                        d.wait()
                    for j in range(kg):
                        pltpu.sync_copy(rows_v.at[pl.ds(j * LW, LW)],
                                        acc_sh.at[idxs_v.at[gi * kg + j]],
                                        add=True)
                return carry

            lax.fori_loop(0, steps, step, 0)
            plsc.subcore_barrier()
            pltpu.sync_copy(acc_sh.at[pl.ds(s * ort, ort)],
                            rows_v.at[pl.ds(0, ort)])
            pltpu.sync_copy(rows_v.at[pl.ds(0, ort)],
                            o.at[pl.ds(s * ort, ort)])
            plsc.subcore_barrier()

        for ci in range(nchunk):
            @pl.when(c == 0)
            def _():
                process(gs[ci], os_[ci])

            @pl.when(c == 1)
            def _():
                process(gs[nchunk + ci], os_[nchunk + ci])

    return pl.kernel(
        body,
        out_type=[jax.ShapeDtypeStruct((n_nodes, fc), F32)] * (2 * nchunk),
        mesh=_sc_mesh(),
        compiler_params=pltpu.CompilerParams(use_tc_tiling_on_sc=False),
        scratch_types=[
            pltpu.VMEM((k, LW), jnp.int32),
            pltpu.VMEM((k, LW), jnp.int32),
            pltpu.VMEM((kg * LW, fc), F32),
            pltpu.VMEM_SHARED((acc_rows, fc), F32),
            pltpu.SemaphoreType.DMA,
        ],
    )


def _dense1_body(x_ref, wm_ref, ws_ref, dh_ref, d1_ref, *out_refs):
    ga_refs = out_refs[0:4]
    gb_refs = out_refs[4:8]
    x = x_ref[...]
    m = jnp.dot(x, wm_ref[...], preferred_element_type=F32,
                 precision=lax.Precision.HIGHEST)
    sg = jnp.dot(x, ws_ref[...], preferred_element_type=F32,
                 precision=lax.Precision.HIGHEST)
    m = jnp.where(m > 0, m, jnp.exp(m) - 1.0)
    sg = jnp.maximum(sg, 0.0)
    att = jnp.exp(-sg)
    dh = dh_ref[...]
    d1 = d1_ref[...]
    ga = dh * (m * att)
    gb = d1 * (sg * att * att)
    for i, r in enumerate(ga_refs):
        r[...] = ga[:, i * 64:(i + 1) * 64]
    for i, r in enumerate(gb_refs):
        r[...] = gb[:, i * 64:(i + 1) * 64]


def _dense2_body(oa_ref, ob_ref, dh_ref, d1_ref, wm_ref, ws_ref,
                 gc0, gc1, s2_ref):
    dh = dh_ref[...]
    d1 = d1_ref[...]
    miu1 = dh * oa_ref[...]
    sig1 = d1 * ob_ref[...]
    m2 = jnp.dot(miu1, wm_ref[...], preferred_element_type=F32,
                 precision=lax.Precision.HIGHEST)
    s2 = jnp.dot(sig1, ws_ref[...], preferred_element_type=F32,
                 precision=lax.Precision.HIGHEST)
    m2 = jnp.where(m2 > 0, m2, jnp.exp(m2) - 1.0)
    s2 = jnp.maximum(s2, 0.0)
    att2 = jnp.exp(-s2)
    gc = dh * (m2 * att2)
    gc0[...] = gc[:, :32]
    gc1[...] = gc[:, 32:]
    s2_ref[...] = s2


def _final_body(oc_ref, dh_ref, s2_ref, eps_ref, out_ref):
    dh = dh_ref[...]
    mean = dh * oc_ref[...]
    o = mean + eps_ref[...] * jnp.sqrt(s2_ref[...] + 1e-8)
    mx = jnp.max(o, axis=1, keepdims=True)
    sh = o - mx
    out_ref[...] = sh - jnp.log(jnp.sum(jnp.exp(sh), axis=1, keepdims=True))


def kernel(x, edge_index, w1_miu, w1_sigma, w2_miu, w2_sigma):
    n, nfeat = x.shape
    nhid = w1_miu.shape[1]
    ncls = w2_miu.shape[1]
    e = edge_index.shape[1]

    src = edge_index[0].astype(jnp.int32)
    dst = edge_index[1].astype(jnp.int32)

    np_ = -(-n // (8 * NS)) * (8 * NS)

    group = LW * NS * 16
    e_pad = -(-e // group) * group
    if e_pad != e:
        src = jnp.concatenate([src, jnp.full((e_pad - e,), n, jnp.int32)])
        dst = jnp.concatenate([dst, jnp.zeros((e_pad - e,), jnp.int32)])
    src2 = src.reshape(-1, LW)
    dst2 = dst.reshape(-1, LW)
    n_rows = e_pad // LW

    x_p = jnp.pad(x, ((0, np_ - n), (0, 0)))
    zeros16 = jnp.zeros((np_, DW), F32)
    zeros64 = jnp.zeros((np_, 64), F32)
    zeros32 = jnp.zeros((np_, 32), F32)
    ones_deg = jnp.ones((8 * LW, DW), F32)

    degp = _build_deg(np_, n_rows, 8)(src2, ones_deg, zeros16)
    deg = degp[0, :, 0:1] + degp[1, :, 0:1]
    dh = jnp.where(deg > 0, lax.rsqrt(deg), 0.0)
    d1 = jnp.where(deg > 0, 1.0 / deg, 0.0)

    grid_r = 1280
    nblk = np_ // grid_r
    row_spec = lambda w: pl.BlockSpec((grid_r, w), lambda i: (i, 0))
    full_spec = lambda a, b: pl.BlockSpec((a, b), lambda i: (0, 0))
    gabs = pl.pallas_call(
        _dense1_body,
        grid=(nblk,),
        in_specs=[row_spec(nfeat), full_spec(nfeat, nhid), full_spec(nfeat, nhid),
                  row_spec(1), row_spec(1)],
        out_specs=[row_spec(64)] * 8,
        out_shape=[jax.ShapeDtypeStruct((np_, 64), F32)] * 8,
    )(x_p, w1_miu, w1_sigma, dh, d1)

    oabs = _build_spmm(np_, n_rows, 64, 4, 8, 8)(*gabs, src2, dst2, zeros64)

    oa = jnp.concatenate(oabs[:4], axis=1)
    ob = jnp.concatenate(oabs[4:], axis=1)

    gc0, gc1, s2 = pl.pallas_call(
        _dense2_body,
        grid=(nblk,),
        in_specs=[row_spec(nhid), row_spec(nhid), row_spec(1), row_spec(1),
                  full_spec(nhid, ncls), full_spec(nhid, ncls)],
        out_specs=[row_spec(32), row_spec(32), row_spec(ncls)],
        out_shape=[jax.ShapeDtypeStruct((np_, 32), F32)] * 2
        + [jax.ShapeDtypeStruct((np_, ncls), F32)],
    )(oa, ob, dh, d1, w2_miu, w2_sigma)

    oc0, oc1 = _build_spmm(np_, n_rows, 32, 1, 8, 8)(
        gc0, gc1, src2, dst2, zeros32
    )
    oc = jnp.concatenate([oc0, oc1], axis=1)

    eps = jax.random.normal(
        jax.random.fold_in(jax.random.key(7), 123), (ncls,), F32
    ).reshape(1, ncls)
    out = pl.pallas_call(
        _final_body,
        grid=(nblk,),
        in_specs=[row_spec(ncls), row_spec(1), row_spec(ncls),
                  full_spec(1, ncls)],
        out_specs=row_spec(ncls),
        out_shape=jax.ShapeDtypeStruct((np_, ncls), F32),
    )(oc, dh, s2, eps)
    return out[:n]
